# baseline (speedup 1.0000x reference)
import jax
import jax.numpy as jnp
from jax.experimental import pallas as pl
from jax.experimental.pallas import tpu as pltpu

_NUM_TOKENS = 16384
_HIDDEN = 2048
_NUM_EXPERTS = 16
_BLOCK = 512
_IDX_PAD = 128


def _router_kernel(x_ref, w_ref, scores_ref, idx_ref):
    x = x_ref[...]
    w = w_ref[...]
    logits = jax.lax.dot_general(
        x, w, (((1,), (1,)), ((), ())), preferred_element_type=jnp.float32
    )

    m = jnp.max(logits, axis=-1, keepdims=True)
    e = jnp.exp(logits - m)
    scores = e / jnp.sum(e, axis=-1, keepdims=True)
    scores_ref[...] = scores

    iota = jax.lax.broadcasted_iota(jnp.int32, logits.shape, 1)
    big = jnp.int32(_NUM_EXPERTS)
    idx0 = jnp.min(jnp.where(logits == m, iota, big), axis=-1, keepdims=True)
    masked = jnp.where(iota == idx0, -jnp.inf, logits)
    m1 = jnp.max(masked, axis=-1, keepdims=True)
    idx1 = jnp.min(jnp.where(masked == m1, iota, big), axis=-1, keepdims=True)

    lane = jax.lax.broadcasted_iota(jnp.int32, (x.shape[0], _IDX_PAD), 1)
    idx_ref[...] = jnp.where(lane == 0, idx0, jnp.where(lane == 1, idx1, 0))


def kernel(hidden_states, weight):
    n_tokens = hidden_states.shape[0]
    grid = n_tokens // _BLOCK
    scores, idx = pl.pallas_call(
        _router_kernel,
        grid=(grid,),
        in_specs=[
            pl.BlockSpec((_BLOCK, _HIDDEN), lambda i: (i, 0)),
            pl.BlockSpec((_NUM_EXPERTS, _HIDDEN), lambda i: (0, 0)),
        ],
        out_specs=[
            pl.BlockSpec((_BLOCK, _NUM_EXPERTS), lambda i: (i, 0)),
            pl.BlockSpec((_BLOCK, _IDX_PAD), lambda i: (i, 0)),
        ],
        out_shape=[
            jax.ShapeDtypeStruct((n_tokens, _NUM_EXPERTS), jnp.float32),
            jax.ShapeDtypeStruct((n_tokens, _IDX_PAD), jnp.int32),
        ],
        compiler_params=pltpu.CompilerParams(
            dimension_semantics=("parallel",),
        ),
    )(hidden_states, weight)
    return scores, idx[:, :2]


# speedup vs baseline: 1.1253x; 1.1253x over previous
import jax
import jax.numpy as jnp
from jax.experimental import pallas as pl
from jax.experimental.pallas import tpu as pltpu

_NUM_TOKENS = 16384
_HIDDEN = 2048
_NUM_EXPERTS = 16
_BLOCK = 1024
_IDX_PAD = 8


def _router_kernel(x_ref, w_ref, scores_ref, idx_ref):
    x = x_ref[...]
    w = w_ref[...]
    logits = jax.lax.dot_general(
        x, w, (((1,), (0,)), ((), ())), preferred_element_type=jnp.float32
    )

    m = jnp.max(logits, axis=-1, keepdims=True)
    e = jnp.exp(logits - m)
    scores = e / jnp.sum(e, axis=-1, keepdims=True)
    scores_ref[...] = scores

    iota = jax.lax.broadcasted_iota(jnp.int32, logits.shape, 1)
    big = jnp.int32(_NUM_EXPERTS)
    idx0 = jnp.min(jnp.where(logits == m, iota, big), axis=-1, keepdims=True)
    masked = jnp.where(iota == idx0, -jnp.inf, logits)
    m1 = jnp.max(masked, axis=-1, keepdims=True)
    idx1 = jnp.min(jnp.where(masked == m1, iota, big), axis=-1, keepdims=True)

    lane = jax.lax.broadcasted_iota(jnp.int32, (x.shape[0], _IDX_PAD), 1)
    idx_ref[...] = jnp.where(lane == 0, idx0, jnp.where(lane == 1, idx1, 0))


def kernel(hidden_states, weight):
    n_tokens = hidden_states.shape[0]
    grid = n_tokens // _BLOCK
    scores, idx = pl.pallas_call(
        _router_kernel,
        grid=(grid,),
        in_specs=[
            pl.BlockSpec((_BLOCK, _HIDDEN), lambda i: (i, 0)),
            pl.BlockSpec((_HIDDEN, _NUM_EXPERTS), lambda i: (0, 0)),
        ],
        out_specs=[
            pl.BlockSpec((_BLOCK, _NUM_EXPERTS), lambda i: (i, 0)),
            pl.BlockSpec((_BLOCK, _IDX_PAD), lambda i: (i, 0)),
        ],
        out_shape=[
            jax.ShapeDtypeStruct((n_tokens, _NUM_EXPERTS), jnp.float32),
            jax.ShapeDtypeStruct((n_tokens, _IDX_PAD), jnp.int32),
        ],
        compiler_params=pltpu.CompilerParams(
            dimension_semantics=("parallel",),
        ),
    )(hidden_states, weight.T)
    return scores, idx[:, :2]
